# baseline (speedup 1.0000x reference)
import jax
import jax.numpy as jnp
from jax import lax
from jax.experimental import pallas as pl
from jax.experimental.pallas import tpu as pltpu
from jax.experimental.pallas import tpu_sc as plsc

N = 1_000_000
L = 16
NT = 16
CH = 65_536
LAST_CH = N - 15 * CH
SUB = CH // 4
NG = CH // 256
NGS = SUB // 256
NG15 = 68
TCAP = 32
MCAP = 128
K = 50
TOP_P = 0.9
NEG = float("-inf")

_f32 = jnp.float32
_i32 = jnp.int32


def _iota():
    return lax.broadcasted_iota(_i32, (L,), 0)


def _lane_f32(v, lane):
    return jnp.max(jnp.where(_iota() == lane, v, jnp.full((L,), NEG, _f32)))


def _lane_i32(v, lane):
    return jnp.max(jnp.where(_iota() == lane, v, jnp.full((L,), -2**31 + 1, _i32)))


def _body(x_hbm, out_hbm, vbuf, gmax, lmall, cval, cidx, zbuf, mval, midx,
          sval, sidx, scatv, scati, stg, sh_in, sh_zero, sh_lmax, sh_cv,
          sh_ci, semh, semld, semz, sems):
    wid = lax.axis_index("s") * 1 + lax.axis_index("c")
    neg16 = jnp.full((L,), NEG, _f32)

    def _slot(s):
        return sh_in.at[pl.ds((wid * 2 + s) * SUB, SUB)]

    @pl.when(wid < 15)
    def _():
        for c in range(2):
            pltpu.async_copy(x_hbm.at[pl.ds(wid * CH + c * SUB, SUB)],
                             _slot(c), semh)

    @pl.when(wid == 15)
    def _():
        pltpu.async_copy(x_hbm.at[pl.ds(15 * CH, LAST_CH)],
                         vbuf.at[pl.ds(0, LAST_CH)], semh)

    z16 = jnp.zeros((L,), _f32)
    for i in range(256):
        zbuf[pl.ds(i * L, L)] = z16
    pltpu.sync_copy(zbuf, sh_zero.at[pl.ds(wid * 4096, 4096)])

    def scan_groups(c, acc):
        def g_body(g, acc):
            m = vbuf[pl.ds(g * 256, L)]
            for j in range(1, 16):
                m = jnp.maximum(m, vbuf[pl.ds(g * 256 + j * L, L)])
            gmax[pl.ds(g * L, L)] = m
            return jnp.maximum(acc, m)
        return lax.fori_loop(c * NGS, (c + 1) * NGS, g_body, acc, unroll=2)

    @pl.when(wid < 15)
    def _():
        lm = neg16

        def wait_h():
            pltpu.make_async_copy(x_hbm.at[pl.ds(0, SUB)],
                                  _slot(0), semh).wait()

        def wait_l():
            pltpu.make_async_copy(_slot(0), vbuf.at[pl.ds(0, SUB)],
                                  semld).wait()

        wait_h()
        pltpu.async_copy(_slot(0), vbuf.at[pl.ds(0, SUB)], semld)
        for c in range(4):
            wait_l()
            if c < 3:
                wait_h()
                pltpu.async_copy(_slot((c + 1) % 2),
                                 vbuf.at[pl.ds((c + 1) * SUB, SUB)], semld)
            if c + 2 < 4:
                pltpu.async_copy(
                    x_hbm.at[pl.ds(wid * CH + (c + 2) * SUB, SUB)],
                    _slot(c % 2), semh)
            lm = scan_groups(c, lm)
        stg[...] = lm

    @pl.when(wid == 15)
    def _():
        for i in range((NG15 * 256 - LAST_CH) // L):
            vbuf[pl.ds(LAST_CH + i * L, L)] = neg16
        pltpu.make_async_copy(x_hbm.at[pl.ds(15 * CH, LAST_CH)],
                              vbuf.at[pl.ds(0, LAST_CH)], semh).wait()

        def g_body(g, acc):
            m = vbuf[pl.ds(g * 256, L)]
            for j in range(1, 16):
                m = jnp.maximum(m, vbuf[pl.ds(g * 256 + j * L, L)])
            gmax[pl.ds(g * L, L)] = m
            return jnp.maximum(acc, m)

        stg[...] = lax.fori_loop(0, NG15, g_body, neg16)

    pltpu.sync_copy(stg, sh_lmax.at[pl.ds(wid * L, L)])
    plsc.subcore_barrier()

    @pl.when(wid < 15)
    def _():
        pltpu.async_copy(sh_zero, out_hbm.at[pl.ds(wid * CH, CH)], semz)

    @pl.when(wid == 15)
    def _():
        for i in range(LAST_CH // 4096):
            pltpu.async_copy(zbuf,
                             out_hbm.at[pl.ds(15 * CH + i * 4096, 4096)],
                             semz)
        pltpu.async_copy(zbuf.at[pl.ds(0, LAST_CH % 4096)],
                         out_hbm.at[pl.ds(15 * CH + (LAST_CH // 4096) * 4096,
                                          LAST_CH % 4096)], semz)

    pltpu.sync_copy(sh_lmax, lmall)
    sgn = jnp.full((L,), -2**31, _i32)
    inv = jnp.full((L,), 0x7FFFFFFF, _i32)
    okeys = []
    for t in range(NT):
        s = plsc.bitcast(lmall[pl.ds(t * L, L)], _i32)
        s = jnp.where(s < jnp.zeros((L,), _i32), s ^ inv, s)
        okeys.append(plsc.bitcast(s ^ sgn, jnp.uint32))
    kv = jnp.full((L,), K, _i32)

    def b_body(b, tacc):
        cand = tacc | plsc.bitcast(
            jnp.full((L,), 1, _i32) << (jnp.full((L,), 31, _i32) - b),
            jnp.uint32)
        cnt = plsc.all_reduce_population_count(okeys[0] >= cand)
        for t in range(1, NT):
            cnt = cnt + plsc.all_reduce_population_count(okeys[t] >= cand)
        return jnp.where(cnt >= kv, cand, tacc)

    tbits = lax.fori_loop(0, 32, b_body, jnp.zeros((L,), jnp.uint32))
    st = plsc.bitcast(tbits, _i32) ^ sgn
    st = jnp.where(st < jnp.zeros((L,), _i32), st ^ inv, st)
    t0v = plsc.bitcast(st, _f32)

    for r in range(TCAP // L):
        cval[pl.ds(r * L, L)] = neg16
        cidx[pl.ds(r * L, L)] = jnp.full((L,), -1, _i32)

    def s_body(b, off):
        gm = gmax[pl.ds(b * (4 * L), L)]
        for u in range(1, 4):
            gm = jnp.maximum(gm, gmax[pl.ds(b * (4 * L) + u * L, L)])

        def hit(offv):
            def inner(j, offv2):
                v = vbuf[pl.ds(b * 1024 + j * L, L)]
                msk = v >= t0v
                idxv = jnp.full((L,), wid * CH + b * 1024 + j * L, _i32) + _iota()
                incl = jnp.cumsum(msk.astype(_i32))
                dest = jnp.minimum(offv2 + incl - 1,
                                   jnp.full((L,), TCAP - 1, _i32))
                plsc.store_scatter(cval, [dest], v, mask=msk)
                plsc.store_scatter(cidx, [dest], idxv, mask=msk)
                return offv2 + plsc.all_reduce_population_count(msk)
            return lax.fori_loop(0, 64, inner, offv)

        return lax.cond(jnp.any(gm >= t0v), hit, lambda o: o, off)

    nb_w = jnp.where(wid == 15, (NG15 + 3) // 4, NG // 4)
    lax.fori_loop(0, nb_w, s_body, jnp.zeros((L,), _i32))
    pltpu.sync_copy(cval, sh_cv.at[pl.ds(wid * TCAP, TCAP)])
    pltpu.sync_copy(cidx, sh_ci.at[pl.ds(wid * TCAP, TCAP)])

    @pl.when(wid < 15)
    def _():
        pltpu.make_async_copy(sh_zero, out_hbm.at[pl.ds(wid * CH, CH)],
                              semz).wait()

    @pl.when(wid == 15)
    def _():
        for i in range(LAST_CH // 4096):
            pltpu.make_async_copy(
                zbuf, out_hbm.at[pl.ds(15 * CH + i * 4096, 4096)],
                semz).wait()
        pltpu.make_async_copy(
            zbuf.at[pl.ds(0, LAST_CH % 4096)],
            out_hbm.at[pl.ds(15 * CH + (LAST_CH // 4096) * 4096,
                             LAST_CH % 4096)], semz).wait()

    plsc.subcore_barrier()

    @pl.when(wid == 0)
    def _():
        z = jnp.zeros((L,), _f32)
        zi = jnp.zeros((L,), _i32)
        for r in range(4):
            scatv[pl.ds(r * L, L)] = z
            scati[pl.ds(r * L, L)] = zi
        pltpu.async_copy(scatv, out_hbm.at[scati], sems).wait()


@jax.jit
def kernel(logits):
    mesh = plsc.VectorSubcoreMesh(core_axis_name="c", subcore_axis_name="s",
                                  num_cores=1)
    f = pl.kernel(
        _body,
        out_type=jax.ShapeDtypeStruct((N,), _f32),
        mesh=mesh,
        scratch_types=[
            pltpu.VMEM((CH,), _f32),
            pltpu.VMEM((NG * L,), _f32),
            pltpu.VMEM((NT * L,), _f32),
            pltpu.VMEM((TCAP,), _f32),
            pltpu.VMEM((TCAP,), _i32),
            pltpu.VMEM((4096,), _f32),
            pltpu.VMEM((NT * TCAP,), _f32),
            pltpu.VMEM((NT * TCAP,), _i32),
            pltpu.VMEM((MCAP,), _f32),
            pltpu.VMEM((MCAP,), _i32),
            pltpu.VMEM((64,), _f32),
            pltpu.VMEM((64,), _i32),
            pltpu.VMEM((L,), _f32),
            pltpu.VMEM_SHARED((15 * 2 * SUB,), _f32),
            pltpu.VMEM_SHARED((CH,), _f32),
            pltpu.VMEM_SHARED((NT * L,), _f32),
            pltpu.VMEM_SHARED((NT * TCAP,), _f32),
            pltpu.VMEM_SHARED((NT * TCAP,), _i32),
            pltpu.SemaphoreType.DMA,
            pltpu.SemaphoreType.DMA,
            pltpu.SemaphoreType.DMA,
            pltpu.SemaphoreType.DMA,
        ],
        compiler_params=pltpu.CompilerParams(needs_layout_passes=False,
                                             disable_bounds_checks=True),
    )
    return f(logits)


# speedup vs baseline: 1.0727x; 1.0727x over previous
import jax
import jax.numpy as jnp
from jax import lax
from jax.experimental import pallas as pl
from jax.experimental.pallas import tpu as pltpu
from jax.experimental.pallas import tpu_sc as plsc

N = 1_000_000
L = 16
NT = 16
CH = 65_536
LAST_CH = N - 15 * CH
SUB = CH // 4
NG = CH // 256
NGS = SUB // 256
NG15 = 68
TCAP = 32
MCAP = 128
K = 50
TOP_P = 0.9
NEG = float("-inf")

_f32 = jnp.float32
_i32 = jnp.int32


def _iota():
    return lax.broadcasted_iota(_i32, (L,), 0)


def _lane_f32(v, lane):
    return jnp.max(jnp.where(_iota() == lane, v, jnp.full((L,), NEG, _f32)))


def _lane_i32(v, lane):
    return jnp.max(jnp.where(_iota() == lane, v, jnp.full((L,), -2**31 + 1, _i32)))


def _body(x_hbm, out_hbm, vbuf, gmax, lmall, cval, cidx, zbuf, mval, midx,
          sval, sidx, scatv, scati, stg, sh_in, sh_zero, sh_lmax, sh_cv,
          sh_ci, semh, semld, semz, sems):
    wid = lax.axis_index("s") * 1 + lax.axis_index("c")
    neg16 = jnp.full((L,), NEG, _f32)

    def _slot(s):
        return sh_in.at[pl.ds((wid * 2 + s) * SUB, SUB)]

    @pl.when(wid < 15)
    def _():
        for c in range(2):
            pltpu.async_copy(x_hbm.at[pl.ds(wid * CH + c * SUB, SUB)],
                             _slot(c), semh)

    @pl.when(wid == 15)
    def _():
        pltpu.async_copy(x_hbm.at[pl.ds(15 * CH, LAST_CH)],
                         vbuf.at[pl.ds(0, LAST_CH)], semh)

    z16 = jnp.zeros((L,), _f32)
    for i in range(256):
        zbuf[pl.ds(i * L, L)] = z16
    pltpu.sync_copy(zbuf, sh_zero.at[pl.ds(wid * 4096, 4096)])

    def scan_groups(c, acc):
        def g_body(g, acc):
            m = vbuf[pl.ds(g * 256, L)]
            for j in range(1, 16):
                m = jnp.maximum(m, vbuf[pl.ds(g * 256 + j * L, L)])
            gmax[pl.ds(g * L, L)] = m
            return jnp.maximum(acc, m)
        return lax.fori_loop(c * NGS, (c + 1) * NGS, g_body, acc, unroll=2)

    @pl.when(wid < 15)
    def _():
        lm = neg16

        def wait_h():
            pltpu.make_async_copy(x_hbm.at[pl.ds(0, SUB)],
                                  _slot(0), semh).wait()

        def wait_l():
            pltpu.make_async_copy(_slot(0), vbuf.at[pl.ds(0, SUB)],
                                  semld).wait()

        wait_h()
        pltpu.async_copy(_slot(0), vbuf.at[pl.ds(0, SUB)], semld)
        for c in range(4):
            wait_l()
            if c < 3:
                wait_h()
                pltpu.async_copy(_slot((c + 1) % 2),
                                 vbuf.at[pl.ds((c + 1) * SUB, SUB)], semld)
            if c + 2 < 4:
                pltpu.async_copy(
                    x_hbm.at[pl.ds(wid * CH + (c + 2) * SUB, SUB)],
                    _slot(c % 2), semh)
            lm = scan_groups(c, lm)
        stg[...] = lm

    @pl.when(wid == 15)
    def _():
        for i in range((NG15 * 256 - LAST_CH) // L):
            vbuf[pl.ds(LAST_CH + i * L, L)] = neg16
        pltpu.make_async_copy(x_hbm.at[pl.ds(15 * CH, LAST_CH)],
                              vbuf.at[pl.ds(0, LAST_CH)], semh).wait()

        def g_body(g, acc):
            m = vbuf[pl.ds(g * 256, L)]
            for j in range(1, 16):
                m = jnp.maximum(m, vbuf[pl.ds(g * 256 + j * L, L)])
            gmax[pl.ds(g * L, L)] = m
            return jnp.maximum(acc, m)

        stg[...] = lax.fori_loop(0, NG15, g_body, neg16)

    pltpu.sync_copy(stg, sh_lmax.at[pl.ds(wid * L, L)])
    plsc.subcore_barrier()

    @pl.when(wid < 15)
    def _():
        pltpu.async_copy(sh_zero, out_hbm.at[pl.ds(wid * CH, CH)], semz)

    @pl.when(wid == 15)
    def _():
        for i in range(LAST_CH // 4096):
            pltpu.async_copy(zbuf,
                             out_hbm.at[pl.ds(15 * CH + i * 4096, 4096)],
                             semz)
        pltpu.async_copy(zbuf.at[pl.ds(0, LAST_CH % 4096)],
                         out_hbm.at[pl.ds(15 * CH + (LAST_CH // 4096) * 4096,
                                          LAST_CH % 4096)], semz)

    pltpu.sync_copy(sh_lmax, lmall)
    sgn = jnp.full((L,), -2**31, _i32)
    inv = jnp.full((L,), 0x7FFFFFFF, _i32)
    okeys = []
    for t in range(NT):
        s = plsc.bitcast(lmall[pl.ds(t * L, L)], _i32)
        s = jnp.where(s < jnp.zeros((L,), _i32), s ^ inv, s)
        okeys.append(plsc.bitcast(s ^ sgn, jnp.uint32))
    kv = jnp.full((L,), K, _i32)

    def b_body(b, tacc):
        cand = tacc | plsc.bitcast(
            jnp.full((L,), 1, _i32) << (jnp.full((L,), 31, _i32) - b),
            jnp.uint32)
        cnt = plsc.all_reduce_population_count(okeys[0] >= cand)
        for t in range(1, NT):
            cnt = cnt + plsc.all_reduce_population_count(okeys[t] >= cand)
        return jnp.where(cnt >= kv, cand, tacc)

    tbits = lax.fori_loop(0, 32, b_body, jnp.zeros((L,), jnp.uint32))
    st = plsc.bitcast(tbits, _i32) ^ sgn
    st = jnp.where(st < jnp.zeros((L,), _i32), st ^ inv, st)
    t0v = plsc.bitcast(st, _f32)

    for r in range(TCAP // L):
        cval[pl.ds(r * L, L)] = neg16
        cidx[pl.ds(r * L, L)] = jnp.full((L,), -1, _i32)

    def s_body(b, off):
        gm = gmax[pl.ds(b * (4 * L), L)]
        for u in range(1, 4):
            gm = jnp.maximum(gm, gmax[pl.ds(b * (4 * L) + u * L, L)])

        def hit(offv):
            def inner(j, offv2):
                v = vbuf[pl.ds(b * 1024 + j * L, L)]
                msk = v >= t0v
                idxv = jnp.full((L,), wid * CH + b * 1024 + j * L, _i32) + _iota()
                incl = jnp.cumsum(msk.astype(_i32))
                dest = jnp.minimum(offv2 + incl - 1,
                                   jnp.full((L,), TCAP - 1, _i32))
                plsc.store_scatter(cval, [dest], v, mask=msk)
                plsc.store_scatter(cidx, [dest], idxv, mask=msk)
                return offv2 + plsc.all_reduce_population_count(msk)
            return lax.fori_loop(0, 64, inner, offv)

        return lax.cond(jnp.any(gm >= t0v), hit, lambda o: o, off)

    nb_w = jnp.where(wid == 15, (NG15 + 3) // 4, NG // 4)
    lax.fori_loop(0, nb_w, s_body, jnp.zeros((L,), _i32))
    pltpu.sync_copy(cval, sh_cv.at[pl.ds(wid * TCAP, TCAP)])
    pltpu.sync_copy(cidx, sh_ci.at[pl.ds(wid * TCAP, TCAP)])

    @pl.when(wid < 15)
    def _():
        pltpu.make_async_copy(sh_zero, out_hbm.at[pl.ds(wid * CH, CH)],
                              semz).wait()

    @pl.when(wid == 15)
    def _():
        for i in range(LAST_CH // 4096):
            pltpu.make_async_copy(
                zbuf, out_hbm.at[pl.ds(15 * CH + i * 4096, 4096)],
                semz).wait()
        pltpu.make_async_copy(
            zbuf.at[pl.ds(0, LAST_CH % 4096)],
            out_hbm.at[pl.ds(15 * CH + (LAST_CH // 4096) * 4096,
                             LAST_CH % 4096)], semz).wait()

    plsc.subcore_barrier()

    @pl.when(wid == 0)
    def _():
        pltpu.sync_copy(sh_cv, mval)
        pltpu.sync_copy(sh_ci, midx)
        for r in range(MCAP // L):
            sval[pl.ds(r * L, L)] = neg16
            sidx[pl.ds(r * L, L)] = jnp.full((L,), 2**30, _i32)

        def c_body(r, offv):
            ii = midx[pl.ds(r * L, L)]
            vv = mval[pl.ds(r * L, L)]
            msk = ii >= jnp.zeros((L,), _i32)
            incl = jnp.cumsum(msk.astype(_i32))
            dest = jnp.minimum(offv + incl - 1,
                               jnp.full((L,), MCAP - 1, _i32))
            plsc.store_scatter(sval, [dest], vv, mask=msk)
            plsc.store_scatter(sidx, [dest], ii, mask=msk)
            return offv + plsc.all_reduce_population_count(msk)

        lax.fori_loop(0, (NT * TCAP) // L, c_body, jnp.zeros((L,), _i32))

        pool0 = tuple(sval[pl.ds(r * L, L)] for r in range(MCAP // L))
        pooli = tuple(sidx[pl.ds(r * L, L)] for r in range(MCAP // L))
        big = jnp.full((L,), 2**30, _i32)

        def e_body(t, carry):
            vs, js = carry
            m = vs[0]
            for r in range(1, MCAP // L):
                m = jnp.maximum(m, vs[r])
            msv = jnp.full((L,), jnp.max(m), _f32)
            cand = js[0]
            for r in range(MCAP // L):
                c = jnp.where(vs[r] == msv, js[r], big)
                cand = c if r == 0 else jnp.minimum(cand, c)
            isv = jnp.full((L,), jnp.min(cand), _i32)
            base = (t // L) * L
            onehot = _iota() == (t - base)
            sv = scatv[pl.ds(base, L)]
            si = scati[pl.ds(base, L)]
            scatv[pl.ds(base, L)] = jnp.where(onehot, msv, sv)
            scati[pl.ds(base, L)] = jnp.where(onehot, isv, si)
            vs = tuple(jnp.where((vs[r] == msv) & (js[r] == isv), neg16, vs[r])
                       for r in range(MCAP // L))
            return vs, js

        lax.fori_loop(0, 64, e_body, (pool0, pooli))

        sv = tuple(scatv[pl.ds(r * L, L)] for r in range(4))
        si = tuple(scati[pl.ds(r * L, L)] for r in range(4))
        kthv = jnp.full((L,), _lane_f32(sv[3], 1), _f32)
        m0v = jnp.full((L,), jnp.max(sv[0]), _f32)
        surv = tuple(s >= kthv for s in sv)
        e = tuple(jnp.where(surv[r], jnp.exp(sv[r] - m0v), jnp.zeros((L,), _f32))
                  for r in range(4))
        z1 = jnp.sum(e[0]) + jnp.sum(e[1]) + jnp.sum(e[2]) + jnp.sum(e[3])
        z1v = jnp.full((L,), z1, _f32)
        p = tuple(e[r] / z1v for r in range(4))
        excl = []
        c = jnp.float32(0.0)
        for r in range(4):
            incl = jnp.cumsum(p[r])
            excl.append(jnp.full((L,), c, _f32) + incl - p[r])
            c = c + jnp.sum(p[r])
        pv = jnp.full((L,), TOP_P, _f32)
        keep = tuple(surv[r] & (excl[r] <= pv) for r in range(4))
        z2 = jnp.float32(0.0)
        for r in range(4):
            z2 = z2 + jnp.sum(jnp.where(keep[r], e[r], jnp.zeros((L,), _f32)))
        z2v = jnp.full((L,), z2, _f32)
        q = tuple(e[r] / z2v for r in range(4))
        q0v = jnp.full((L,), _lane_f32(q[0], 0), _f32)
        i0v = jnp.full((L,), _lane_i32(si[0], 0), _i32)
        for r in range(4):
            scatv[pl.ds(r * L, L)] = jnp.where(keep[r], q[r], q0v)
            scati[pl.ds(r * L, L)] = jnp.where(keep[r], si[r], i0v)
        pltpu.async_copy(scatv, out_hbm.at[scati], sems).wait()


@jax.jit
def kernel(logits):
    mesh = plsc.VectorSubcoreMesh(core_axis_name="c", subcore_axis_name="s",
                                  num_cores=1)
    f = pl.kernel(
        _body,
        out_type=jax.ShapeDtypeStruct((N,), _f32),
        mesh=mesh,
        scratch_types=[
            pltpu.VMEM((CH,), _f32),
            pltpu.VMEM((NG * L,), _f32),
            pltpu.VMEM((NT * L,), _f32),
            pltpu.VMEM((TCAP,), _f32),
            pltpu.VMEM((TCAP,), _i32),
            pltpu.VMEM((4096,), _f32),
            pltpu.VMEM((NT * TCAP,), _f32),
            pltpu.VMEM((NT * TCAP,), _i32),
            pltpu.VMEM((MCAP,), _f32),
            pltpu.VMEM((MCAP,), _i32),
            pltpu.VMEM((64,), _f32),
            pltpu.VMEM((64,), _i32),
            pltpu.VMEM((L,), _f32),
            pltpu.VMEM_SHARED((15 * 2 * SUB,), _f32),
            pltpu.VMEM_SHARED((CH,), _f32),
            pltpu.VMEM_SHARED((NT * L,), _f32),
            pltpu.VMEM_SHARED((NT * TCAP,), _f32),
            pltpu.VMEM_SHARED((NT * TCAP,), _i32),
            pltpu.SemaphoreType.DMA,
            pltpu.SemaphoreType.DMA,
            pltpu.SemaphoreType.DMA,
            pltpu.SemaphoreType.DMA,
        ],
        compiler_params=pltpu.CompilerParams(needs_layout_passes=False,
                                             disable_bounds_checks=True),
    )
    return f(logits)
